# baseline (speedup 1.0000x reference)
import functools

import jax
import jax.numpy as jnp
from jax import lax
from jax.experimental import pallas as pl
from jax.experimental.pallas import tpu as pltpu
from jax.experimental.pallas import tpu_sc as plsc

_SEQ = 2048
_EMBED = 768
_BATCH = 4
_NC = 2
_NS = 16
_NW = _NC * _NS
_S_PER_W = _SEQ // _NW
_CHUNK = 16
_NCH = _S_PER_W // _CHUNK
_NTOT = _BATCH * _NCH
_NBUF = 6
_LANES = 16
_VREGS = _EMBED // _LANES


def _body(ids_hbm, wte_hbm, wpe_hbm, out_hbm, idx_v, pos_v, bufs, gsems,
          osems, psem, isem):
    cid = lax.axis_index("c")
    sid = lax.axis_index("s")
    wid = sid * _NC + cid
    s0 = wid * _S_PER_W

    id_copies = [
        pltpu.async_copy(ids_hbm.at[pl.ds(b * _SEQ + s0, _S_PER_W)],
                         idx_v.at[b], isem)
        for b in range(_BATCH)
    ]
    pos_copy = pltpu.async_copy(wpe_hbm.at[pl.ds(s0, _S_PER_W)], pos_v, psem)
    for c in id_copies:
        c.wait()

    def fire_gather(k):
        b, c = divmod(k, _NCH)
        i = k % _NBUF
        idx = idx_v.at[b, pl.ds(c * _CHUNK, _CHUNK)]
        return pltpu.async_copy(wte_hbm.at[idx], bufs[i], gsems[i])

    def fire_out(k):
        b, c = divmod(k, _NCH)
        i = k % _NBUF
        dst = out_hbm.at[pl.ds(b * _SEQ + s0 + c * _CHUNK, _CHUNK)]
        return pltpu.async_copy(bufs[i], dst, osems[i])

    gcopies = [None] * _NTOT
    ocopies = [None] * _NTOT
    for k in range(_NBUF - 1):
        gcopies[k] = fire_gather(k)
    pos_copy.wait()

    for k in range(_NTOT):
        i = k % _NBUF
        if k + _NBUF - 1 < _NTOT:
            if k >= 1:
                ocopies[k - 1].wait()
            gcopies[k + _NBUF - 1] = fire_gather(k + _NBUF - 1)
        gcopies[k].wait()
        b, c = divmod(k, _NCH)
        buf = bufs[i]

        def row_body(r, _, buf=buf, c=c):
            pr = c * _CHUNK + r
            for j in range(_VREGS):
                sl = pl.ds(j * _LANES, _LANES)
                plsc.addupdate(buf.at[r, sl], pos_v[pr, sl])
            return 0

        lax.fori_loop(0, _CHUNK, row_body, 0)
        ocopies[k] = fire_out(k)

    for k in range(_NTOT - _NBUF, _NTOT):
        ocopies[k].wait()


_emb = functools.partial(
    pl.kernel,
    out_type=jax.ShapeDtypeStruct((_BATCH * _SEQ, _EMBED), jnp.float32),
    mesh=plsc.VectorSubcoreMesh(core_axis_name="c", subcore_axis_name="s"),
    scratch_types=[
        pltpu.VMEM((_BATCH, _S_PER_W), jnp.int32),
        pltpu.VMEM((_S_PER_W, _EMBED), jnp.float32),
        [pltpu.VMEM((_CHUNK, _EMBED), jnp.float32) for _ in range(_NBUF)],
        [pltpu.SemaphoreType.DMA for _ in range(_NBUF)],
        [pltpu.SemaphoreType.DMA for _ in range(_NBUF)],
        pltpu.SemaphoreType.DMA,
        pltpu.SemaphoreType.DMA,
    ],
)(_body)


@jax.jit
def kernel(input_ids, token_embeddings, position_embeddings):
    ids = input_ids.reshape(-1).astype(jnp.int32)
    out = _emb(ids, token_embeddings, position_embeddings)
    return out.reshape(_BATCH, _SEQ, _EMBED)


# speedup vs baseline: 1.2123x; 1.2123x over previous
import functools

import jax
import jax.numpy as jnp
from jax import lax
from jax.experimental import pallas as pl
from jax.experimental.pallas import tpu as pltpu
from jax.experimental.pallas import tpu_sc as plsc

_SEQ = 2048
_EMBED = 768
_BATCH = 4
_NC = 2
_NS = 16
_NW = _NC * _NS
_S_PER_W = _SEQ // _NW
_CHUNK = 32
_NCH = _S_PER_W // _CHUNK
_NTOT = _BATCH * _NCH
_NBUF = 3
_LANES = 16
_VREGS = _EMBED // _LANES


def _body(ids_hbm, wte_hbm, wpe_hbm, out_hbm, idx_v, pos_v, bufs, gsems,
          osems, psem, isem):
    cid = lax.axis_index("c")
    sid = lax.axis_index("s")
    wid = sid * _NC + cid
    s0 = wid * _S_PER_W

    id_copies = [
        pltpu.async_copy(ids_hbm.at[pl.ds(b * _SEQ + s0, _S_PER_W)],
                         idx_v.at[b], isem)
        for b in range(_BATCH)
    ]
    pos_copy = pltpu.async_copy(wpe_hbm.at[pl.ds(s0, _S_PER_W)], pos_v, psem)
    for c in id_copies:
        c.wait()

    def fire_gather(k):
        b, c = divmod(k, _NCH)
        i = k % _NBUF
        idx = idx_v.at[b, pl.ds(c * _CHUNK, _CHUNK)]
        return pltpu.async_copy(wte_hbm.at[idx], bufs[i], gsems[i])

    def fire_out(k):
        b, c = divmod(k, _NCH)
        i = k % _NBUF
        dst = out_hbm.at[pl.ds(b * _SEQ + s0 + c * _CHUNK, _CHUNK)]
        return pltpu.async_copy(bufs[i], dst, osems[i])

    gcopies = [None] * _NTOT
    ocopies = [None] * _NTOT
    for k in range(_NBUF - 1):
        gcopies[k] = fire_gather(k)
    pos_copy.wait()

    for k in range(_NTOT):
        i = k % _NBUF
        if k + _NBUF - 1 < _NTOT:
            if k >= 1:
                ocopies[k - 1].wait()
            gcopies[k + _NBUF - 1] = fire_gather(k + _NBUF - 1)
        gcopies[k].wait()
        b, c = divmod(k, _NCH)
        buf = bufs[i]

        @plsc.parallel_loop(0, _CHUNK, 1, unroll=2)
        def row_body(r, buf=buf, c=c):
            pr = c * _CHUNK + r
            for j in range(_VREGS):
                sl = pl.ds(j * _LANES, _LANES)
                plsc.addupdate(buf.at[r, sl], pos_v[pr, sl])

        ocopies[k] = fire_out(k)

    for k in range(_NTOT - _NBUF, _NTOT):
        ocopies[k].wait()


_emb = functools.partial(
    pl.kernel,
    out_type=jax.ShapeDtypeStruct((_BATCH * _SEQ, _EMBED), jnp.float32),
    mesh=plsc.VectorSubcoreMesh(core_axis_name="c", subcore_axis_name="s"),
    scratch_types=[
        pltpu.VMEM((_BATCH, _S_PER_W), jnp.int32),
        pltpu.VMEM((_S_PER_W, _EMBED), jnp.float32),
        [pltpu.VMEM((_CHUNK, _EMBED), jnp.float32) for _ in range(_NBUF)],
        [pltpu.SemaphoreType.DMA for _ in range(_NBUF)],
        [pltpu.SemaphoreType.DMA for _ in range(_NBUF)],
        pltpu.SemaphoreType.DMA,
        pltpu.SemaphoreType.DMA,
    ],
)(_body)


@jax.jit
def kernel(input_ids, token_embeddings, position_embeddings):
    ids = input_ids.reshape(-1).astype(jnp.int32)
    out = _emb(ids, token_embeddings, position_embeddings)
    return out.reshape(_BATCH, _SEQ, _EMBED)
